# Initial kernel scaffold; baseline (speedup 1.0000x reference)
import jax
import jax.numpy as jnp
from jax.experimental import pallas as pl


def kernel(x, edge_index, W0, b0, W1, b1, W2, b2, W3, b3, W4, b4, W5, b5):
    raise NotImplementedError("write your pallas kernel here")


# speedup vs baseline: 6.5351x; 6.5351x over previous
import functools

import jax
import jax.numpy as jnp
from jax import lax
from jax.experimental import pallas as pl
from jax.experimental.pallas import tpu as pltpu
from jax.experimental.pallas import tpu_sc as plsc

_N, _E, _D = 10000, 320000, 128
_NC, _NS = 2, 16
_NTILES = _NC * _NS
_EPT = _E // _NTILES
_K = 80
_NCHUNK = _EPT // _K
_ZROWS = _N // _NS


def _make_aggregate():
  mesh = plsc.VectorSubcoreMesh(core_axis_name="c", subcore_axis_name="s")

  @functools.partial(
      pl.kernel,
      mesh=mesh,
      out_type=jax.ShapeDtypeStruct((_NC, _NS, _ZROWS, _D), jnp.float32),
      scratch_types=[
          pltpu.VMEM((_NCHUNK, _K), jnp.int32),
          pltpu.VMEM((_NCHUNK, _K), jnp.int32),
          pltpu.VMEM((_K, _D), jnp.float32),
          pltpu.VMEM_SHARED((_N, _D), jnp.float32),
          pltpu.SemaphoreType.DMA,
      ],
  )
  def agg_kernel(h_hbm, src_hbm, dst_hbm, zero_hbm, out_hbm,
                 src_v, dst_v, rows_v, agg_sh, sem):
    c = lax.axis_index("c")
    s = lax.axis_index("s")
    wid = c * _NS + s
    pltpu.sync_copy(src_hbm.at[wid], src_v)
    pltpu.sync_copy(dst_hbm.at[wid], dst_v)
    pltpu.sync_copy(zero_hbm, agg_sh.at[pl.ds(s * _ZROWS, _ZROWS)])
    plsc.subcore_barrier()

    def body(j, carry):
      pltpu.async_copy(h_hbm.at[src_v.at[j]], rows_v, sem).wait()
      pltpu.sync_copy(rows_v, agg_sh.at[dst_v.at[j]], add=True)
      return carry

    lax.fori_loop(0, _NCHUNK, body, 0)
    plsc.subcore_barrier()
    pltpu.sync_copy(agg_sh.at[pl.ds(s * _ZROWS, _ZROWS)], out_hbm.at[c, s])

  return agg_kernel


_AGGREGATE = _make_aggregate()

_BLK = 400


def _mlp_body(p_ref, h_ref, wa_ref, ba_ref, wb_ref, bb_ref, o_ref):
  t = p_ref[0] + p_ref[1] + h_ref[...]
  t = jnp.dot(t, wa_ref[...], preferred_element_type=jnp.float32,
              precision=lax.Precision.HIGHEST) + ba_ref[...]
  t = jnp.maximum(t, 0.0)
  t = jnp.dot(t, wb_ref[...], preferred_element_type=jnp.float32,
              precision=lax.Precision.HIGHEST) + bb_ref[...]
  o_ref[...] = jnp.maximum(t, 0.0)


def _mlp(parts, h, wa_t, ba, wb_t, bb):
  return pl.pallas_call(
      _mlp_body,
      grid=(_N // _BLK,),
      in_specs=[
          pl.BlockSpec((_NC, _BLK, _D), lambda i: (0, i, 0)),
          pl.BlockSpec((_BLK, _D), lambda i: (i, 0)),
          pl.BlockSpec((_D, _D), lambda i: (0, 0)),
          pl.BlockSpec((1, _D), lambda i: (0, 0)),
          pl.BlockSpec((_D, _D), lambda i: (0, 0)),
          pl.BlockSpec((1, _D), lambda i: (0, 0)),
      ],
      out_specs=pl.BlockSpec((_BLK, _D), lambda i: (i, 0)),
      out_shape=jax.ShapeDtypeStruct((_N, _D), jnp.float32),
  )(parts, h, wa_t, ba, wb_t, bb)


def kernel(x, edge_index, W0, b0, W1, b1, W2, b2, W3, b3, W4, b4, W5, b5):
  src = edge_index[0].reshape(_NTILES, _NCHUNK, _K)
  dst = edge_index[1].reshape(_NTILES, _NCHUNK, _K)
  zeros_blk = jnp.zeros((_ZROWS, _D), jnp.float32)
  h = x
  for wa, ba, wb, bb in ((W0, b0, W1, b1), (W2, b2, W3, b3), (W4, b4, W5, b5)):
    parts = _AGGREGATE(h, src, dst, zeros_blk).reshape(_NC, _N, _D)
    h = _mlp(parts, h, wa.T, ba.reshape(1, _D), wb.T, bb.reshape(1, _D))
  return h


# speedup vs baseline: 10.0024x; 1.5306x over previous
import functools

import jax
import jax.numpy as jnp
from jax import lax
from jax.experimental import pallas as pl
from jax.experimental.pallas import tpu as pltpu
from jax.experimental.pallas import tpu_sc as plsc

_N, _E, _D = 10000, 320000, 128
_NC, _NS = 2, 16
_NTILES = _NC * _NS
_EPT = _E // _NTILES
_K = 80
_NCHUNK = _EPT // _K
_ZROWS = _N // _NS


def _make_aggregate():
  mesh = plsc.VectorSubcoreMesh(core_axis_name="c", subcore_axis_name="s")

  @functools.partial(
      pl.kernel,
      mesh=mesh,
      out_type=jax.ShapeDtypeStruct((_NC, _NS, _ZROWS, _D), jnp.float32),
      scratch_types=[
          pltpu.VMEM((_EPT,), jnp.int32),
          pltpu.VMEM((_NCHUNK, _K), jnp.int32),
          pltpu.VMEM((_K, _D), jnp.float32),
          pltpu.VMEM((_K, _D), jnp.float32),
          pltpu.VMEM_SHARED((_N, _D), jnp.float32),
          pltpu.SemaphoreType.DMA,
          pltpu.SemaphoreType.DMA,
      ],
  )
  def agg_kernel(h_hbm, src_hbm, dst_hbm, zero_hbm, out_hbm,
                 src_v, dst_v, rows0_v, rows1_v, agg_sh, sem0, sem1):
    c = lax.axis_index("c")
    s = lax.axis_index("s")
    wid = c * _NS + s
    pltpu.sync_copy(src_hbm.at[wid], src_v)
    pltpu.sync_copy(dst_hbm.at[wid], dst_v)
    pltpu.sync_copy(zero_hbm, agg_sh.at[pl.ds(s * _ZROWS, _ZROWS)])
    plsc.subcore_barrier()

    def gather(j, buf, sem):
      pltpu.async_copy(h_hbm.at[src_v.at[pl.ds(j * _K, _K)]], buf, sem)

    def drain(buf, sem):
      pltpu.make_async_copy(h_hbm.at[src_v.at[pl.ds(0, _K)]], buf, sem).wait()

    def scatter_add(j, buf):
      pltpu.sync_copy(buf, agg_sh.at[dst_v.at[j]], add=True)

    gather(0, rows0_v, sem0)
    gather(1, rows1_v, sem1)

    def pair(i, carry):
      j = 2 * i
      drain(rows0_v, sem0)
      scatter_add(j, rows0_v)
      gather(j + 2, rows0_v, sem0)
      drain(rows1_v, sem1)
      scatter_add(j + 1, rows1_v)
      gather(j + 3, rows1_v, sem1)
      return carry

    lax.fori_loop(0, (_NCHUNK - 3) // 2, pair, 0)
    drain(rows0_v, sem0)
    scatter_add(_NCHUNK - 3, rows0_v)
    gather(_NCHUNK - 1, rows0_v, sem0)
    drain(rows1_v, sem1)
    scatter_add(_NCHUNK - 2, rows1_v)
    drain(rows0_v, sem0)
    scatter_add(_NCHUNK - 1, rows0_v)
    plsc.subcore_barrier()
    pltpu.sync_copy(agg_sh.at[pl.ds(s * _ZROWS, _ZROWS)], out_hbm.at[c, s])

  return agg_kernel


_AGGREGATE = _make_aggregate()

_BLK = 400


def _mlp_body(p_ref, h_ref, wa_ref, ba_ref, wb_ref, bb_ref, o_ref):
  t = p_ref[0] + p_ref[1] + h_ref[...]
  t = jnp.dot(t, wa_ref[...], preferred_element_type=jnp.float32,
              precision=lax.Precision.HIGHEST) + ba_ref[...]
  t = jnp.maximum(t, 0.0)
  t = jnp.dot(t, wb_ref[...], preferred_element_type=jnp.float32,
              precision=lax.Precision.HIGHEST) + bb_ref[...]
  o_ref[...] = jnp.maximum(t, 0.0)


def _mlp(parts, h, wa_t, ba, wb_t, bb):
  return pl.pallas_call(
      _mlp_body,
      grid=(_N // _BLK,),
      in_specs=[
          pl.BlockSpec((_NC, _BLK, _D), lambda i: (0, i, 0)),
          pl.BlockSpec((_BLK, _D), lambda i: (i, 0)),
          pl.BlockSpec((_D, _D), lambda i: (0, 0)),
          pl.BlockSpec((1, _D), lambda i: (0, 0)),
          pl.BlockSpec((_D, _D), lambda i: (0, 0)),
          pl.BlockSpec((1, _D), lambda i: (0, 0)),
      ],
      out_specs=pl.BlockSpec((_BLK, _D), lambda i: (i, 0)),
      out_shape=jax.ShapeDtypeStruct((_N, _D), jnp.float32),
  )(parts, h, wa_t, ba, wb_t, bb)


def kernel(x, edge_index, W0, b0, W1, b1, W2, b2, W3, b3, W4, b4, W5, b5):
  src = edge_index[0].reshape(_NTILES, _EPT)
  dst = edge_index[1].reshape(_NTILES, _NCHUNK, _K)
  zeros_blk = jnp.zeros((_ZROWS, _D), jnp.float32)
  h = x
  for wa, ba, wb, bb in ((W0, b0, W1, b1), (W2, b2, W3, b3), (W4, b4, W5, b5)):
    parts = _AGGREGATE(h, src, dst, zeros_blk).reshape(_NC, _N, _D)
    h = _mlp(parts, h, wa.T, ba.reshape(1, _D), wb.T, bb.reshape(1, _D))
  return h


# speedup vs baseline: 10.1248x; 1.0122x over previous
import functools

import jax
import jax.numpy as jnp
from jax import lax
from jax.experimental import pallas as pl
from jax.experimental.pallas import tpu as pltpu
from jax.experimental.pallas import tpu_sc as plsc

_N, _E, _D = 10000, 320000, 128
_NC, _NS = 2, 16
_NTILES = _NC * _NS
_EPT = _E // _NTILES
_K = 80
_NCHUNK = _EPT // _K
_ZROWS = _N // _NS


def _make_aggregate():
  mesh = plsc.VectorSubcoreMesh(core_axis_name="c", subcore_axis_name="s")

  @functools.partial(
      pl.kernel,
      mesh=mesh,
      out_type=jax.ShapeDtypeStruct((_NC, _NS, _ZROWS, _D), jnp.float32),
      scratch_types=[
          pltpu.VMEM((_EPT,), jnp.int32),
          pltpu.VMEM((_NCHUNK, _K), jnp.int32),
          pltpu.VMEM((_K, _D), jnp.float32),
          pltpu.VMEM((_K, _D), jnp.float32),
          pltpu.VMEM_SHARED((_N, _D), jnp.float32),
          pltpu.SemaphoreType.DMA,
          pltpu.SemaphoreType.DMA,
          pltpu.SemaphoreType.DMA,
      ],
  )
  def agg_kernel(h_hbm, src_hbm, dst_hbm, zero_hbm, out_hbm,
                 src_v, dst_v, rows0_v, rows1_v, agg_sh, sem0, sem1, semz):
    c = lax.axis_index("c")
    s = lax.axis_index("s")
    wid = c * _NS + s
    src_stage = pltpu.async_copy(src_hbm.at[wid], src_v, sem0)
    dst_stage = pltpu.async_copy(dst_hbm.at[wid], dst_v, semz)
    zero_stage = pltpu.async_copy(zero_hbm, agg_sh.at[pl.ds(s * _ZROWS, _ZROWS)],
                                  semz)
    src_stage.wait()

    def gather(j, buf, sem):
      pltpu.async_copy(h_hbm.at[src_v.at[pl.ds(j * _K, _K)]], buf, sem)

    def drain(buf, sem):
      pltpu.make_async_copy(h_hbm.at[src_v.at[pl.ds(0, _K)]], buf, sem).wait()

    def scatter_add(j, buf):
      pltpu.sync_copy(buf, agg_sh.at[dst_v.at[j]], add=True)

    gather(0, rows0_v, sem0)
    gather(1, rows1_v, sem1)
    dst_stage.wait()
    zero_stage.wait()
    plsc.subcore_barrier()

    def pair(i, carry):
      j = 2 * i
      drain(rows0_v, sem0)
      scatter_add(j, rows0_v)
      gather(j + 2, rows0_v, sem0)
      drain(rows1_v, sem1)
      scatter_add(j + 1, rows1_v)
      gather(j + 3, rows1_v, sem1)
      return carry

    lax.fori_loop(0, (_NCHUNK - 3) // 2, pair, 0)
    drain(rows0_v, sem0)
    scatter_add(_NCHUNK - 3, rows0_v)
    gather(_NCHUNK - 1, rows0_v, sem0)
    drain(rows1_v, sem1)
    scatter_add(_NCHUNK - 2, rows1_v)
    drain(rows0_v, sem0)
    scatter_add(_NCHUNK - 1, rows0_v)
    plsc.subcore_barrier()
    pltpu.sync_copy(agg_sh.at[pl.ds(s * _ZROWS, _ZROWS)], out_hbm.at[c, s])

  return agg_kernel


_AGGREGATE = _make_aggregate()

_BLK = 400


def _mlp_body(p_ref, h_ref, wa_ref, ba_ref, wb_ref, bb_ref, o_ref):
  t = p_ref[0] + p_ref[1] + h_ref[...]
  t = jnp.dot(t, wa_ref[...], preferred_element_type=jnp.float32,
              precision=lax.Precision.HIGHEST) + ba_ref[...]
  t = jnp.maximum(t, 0.0)
  t = jnp.dot(t, wb_ref[...], preferred_element_type=jnp.float32,
              precision=lax.Precision.HIGHEST) + bb_ref[...]
  o_ref[...] = jnp.maximum(t, 0.0)


def _mlp(parts, h, wa_t, ba, wb_t, bb):
  return pl.pallas_call(
      _mlp_body,
      grid=(_N // _BLK,),
      in_specs=[
          pl.BlockSpec((_NC, _BLK, _D), lambda i: (0, i, 0)),
          pl.BlockSpec((_BLK, _D), lambda i: (i, 0)),
          pl.BlockSpec((_D, _D), lambda i: (0, 0)),
          pl.BlockSpec((1, _D), lambda i: (0, 0)),
          pl.BlockSpec((_D, _D), lambda i: (0, 0)),
          pl.BlockSpec((1, _D), lambda i: (0, 0)),
      ],
      out_specs=pl.BlockSpec((_BLK, _D), lambda i: (i, 0)),
      out_shape=jax.ShapeDtypeStruct((_N, _D), jnp.float32),
  )(parts, h, wa_t, ba, wb_t, bb)


def kernel(x, edge_index, W0, b0, W1, b1, W2, b2, W3, b3, W4, b4, W5, b5):
  src = edge_index[0].reshape(_NTILES, _EPT)
  dst = edge_index[1].reshape(_NTILES, _NCHUNK, _K)
  zeros_blk = jnp.zeros((_ZROWS, _D), jnp.float32)
  h = x
  for wa, ba, wb, bb in ((W0, b0, W1, b1), (W2, b2, W3, b3), (W4, b4, W5, b5)):
    parts = _AGGREGATE(h, src, dst, zeros_blk).reshape(_NC, _N, _D)
    h = _mlp(parts, h, wa.T, ba.reshape(1, _D), wb.T, bb.reshape(1, _D))
  return h
